# Initial kernel scaffold; baseline (speedup 1.0000x reference)
import jax
import jax.numpy as jnp
from jax.experimental import pallas as pl


def kernel(p, x, o, Wq, bq, Wk, bk, Wv, bv, Wp1, bp1, gp, betap, Wp2, bp2, gw1, betaw1, Ww1, bw1, gw2, betaw2, Ww2, bw2):
    raise NotImplementedError("write your pallas kernel here")


# speedup vs baseline: 8.0073x; 8.0073x over previous
import functools

import jax
import jax.numpy as jnp
from jax import lax
from jax.experimental import pallas as pl
from jax.experimental.pallas import tpu as pltpu
from jax.experimental.pallas import tpu_sc as plsc

_N = 8192
_NB = 4
_SEG = _N // _NB
_CIN = 128
_COUT = 128
_HEADS = 8
_MID = _COUT // _HEADS
_NS = 16
_EPS = 1e-5

_QKV_BR = 1024


def _qkv_body(x_ref, w_ref, b_ref, xq_ref, xk_ref, xv_ref):
    acc = jnp.dot(x_ref[...], w_ref[...], preferred_element_type=jnp.float32)
    acc = acc + b_ref[...]
    xq_ref[...] = acc[:, 0:_COUT]
    xk_ref[...] = acc[:, _COUT:2 * _COUT]
    xv_ref[...] = acc[:, 2 * _COUT:3 * _COUT]


def _qkv(x, wall, ball):
    n = x.shape[0]
    grid = (n // _QKV_BR,)
    return pl.pallas_call(
        _qkv_body,
        grid=grid,
        in_specs=[
            pl.BlockSpec((_QKV_BR, _CIN), lambda i: (i, 0)),
            pl.BlockSpec((_CIN, 3 * _COUT), lambda i: (0, 0)),
            pl.BlockSpec((1, 3 * _COUT), lambda i: (0, 0)),
        ],
        out_specs=[
            pl.BlockSpec((_QKV_BR, _COUT), lambda i: (i, 0)),
            pl.BlockSpec((_QKV_BR, _COUT), lambda i: (i, 0)),
            pl.BlockSpec((_QKV_BR, _COUT), lambda i: (i, 0)),
        ],
        out_shape=[jax.ShapeDtypeStruct((n, _COUT), jnp.float32)] * 3,
    )(x, wall, ball)


_KNN_BR = 256


def _knn_body(u_ref, vt_ref, idx_ref):
    s = pl.program_id(0)
    d = jnp.dot(u_ref[...], vt_ref[...], preferred_element_type=jnp.float32,
                precision=lax.Precision.HIGHEST)
    iota = lax.broadcasted_iota(jnp.int32, (_KNN_BR, _SEG), 1)
    cols = []
    for _ in range(_NS):
        m = jnp.min(d, axis=1, keepdims=True)
        am = jnp.min(jnp.where(d <= m, iota, _SEG), axis=1, keepdims=True)
        cols.append(am)
        d = jnp.where(iota == am, jnp.float32(3.0e38), d)
    idx_ref[...] = jnp.concatenate(cols, axis=1) + s * _SEG


def _knn(u, vt):
    grid = (_NB, _SEG // _KNN_BR)
    return pl.pallas_call(
        _knn_body,
        grid=grid,
        in_specs=[
            pl.BlockSpec((_KNN_BR, 8), lambda s, r: (s * (_SEG // _KNN_BR) + r, 0)),
            pl.BlockSpec((8, _SEG), lambda s, r: (0, s)),
        ],
        out_specs=pl.BlockSpec(
            (_KNN_BR, _NS), lambda s, r: (s * (_SEG // _KNN_BR) + r, 0)),
        out_shape=jax.ShapeDtypeStruct((_N, _NS), jnp.int32),
    )(u, vt)


_GCH = 128


def _sc_gather(ppad, xk, xv, idx_flat):
    b_total = idx_flat.shape[0]
    nw = 32
    b_per_w = b_total // nw
    nchunks = b_per_w // _GCH
    mesh = plsc.VectorSubcoreMesh(core_axis_name="c", subcore_axis_name="s")

    @functools.partial(
        pl.kernel,
        out_type=[
            jax.ShapeDtypeStruct((b_total, 16), jnp.float32),
            jax.ShapeDtypeStruct((b_total, _COUT), jnp.float32),
            jax.ShapeDtypeStruct((b_total, _COUT), jnp.float32),
        ],
        mesh=mesh,
        compiler_params=pltpu.CompilerParams(use_tc_tiling_on_sc=False),
        scratch_types=[
            pltpu.VMEM((_GCH,), jnp.int32),
            pltpu.VMEM((_GCH, 16), jnp.float32),
            pltpu.VMEM((_GCH, _COUT), jnp.float32),
            pltpu.VMEM((_GCH, _COUT), jnp.float32),
            pltpu.SemaphoreType.DMA,
            pltpu.SemaphoreType.DMA,
            pltpu.SemaphoreType.DMA,
        ],
    )
    def gather_k(ppad_hbm, xk_hbm, xv_hbm, idx_hbm, pg_out, kg_out, vg_out,
                 idx_v, pbuf, kbuf, vbuf, sem_p, sem_k, sem_v):
        wid = lax.axis_index("s") * 2 + lax.axis_index("c")
        base = pl.multiple_of(wid * b_per_w, 8)

        def chunk(ci, carry):
            off = pl.multiple_of(base + ci * _GCH, 8)
            pltpu.sync_copy(idx_hbm.at[pl.ds(off, _GCH)], idx_v)
            cp = pltpu.async_copy(ppad_hbm.at[idx_v], pbuf, sem_p)
            ck = pltpu.async_copy(xk_hbm.at[idx_v], kbuf, sem_k)
            cv = pltpu.async_copy(xv_hbm.at[idx_v], vbuf, sem_v)
            cp.wait()
            ck.wait()
            cv.wait()
            pltpu.sync_copy(pbuf, pg_out.at[pl.ds(off, _GCH)])
            pltpu.sync_copy(kbuf, kg_out.at[pl.ds(off, _GCH)])
            pltpu.sync_copy(vbuf, vg_out.at[pl.ds(off, _GCH)])
            return carry

        lax.fori_loop(0, nchunks, chunk, 0)

    return gather_k(ppad, xk, xv, idx_flat)


_MLP_BR = 256


def _mlp_body(xq_ref, pp_ref, pg_ref, kg_ref, vg_ref,
              w1p_ref, bpp1_ref, m3_ref, gpp_ref, bpp_ref, w2p_ref, bp2_ref,
              m16_ref, g1_ref, b1_ref, bd1_ref, bw1_ref,
              g2_ref, b2_ref, bd2m_ref, c2_ref, out_ref):
    xq = xq_ref[...]
    pp = pp_ref[...]
    w1p = w1p_ref[...]
    m3 = m3_ref[...]
    w2p = w2p_ref[...]
    m16 = m16_ref[...]
    bd1 = bd1_ref[...]
    bd2m = bd2m_ref[...]

    wms = []
    vals = []
    for k in range(_NS):
        rel = pg_ref[k] - pp
        a = jnp.dot(rel, w1p, preferred_element_type=jnp.float32) + bpp1_ref[...]
        mu = jnp.dot(a, m3, preferred_element_type=jnp.float32)
        va = jnp.dot(a * a, m3, preferred_element_type=jnp.float32) - mu * mu
        h = (a - mu) * lax.rsqrt(va + _EPS) * gpp_ref[...] + bpp_ref[...]
        h = jnp.maximum(h, 0.0)
        pr = jnp.dot(h, w2p, preferred_element_type=jnp.float32) + bp2_ref[...]

        r = kg_ref[k] + pr - xq
        mu1 = jnp.dot(r, m16, preferred_element_type=jnp.float32)
        va1 = jnp.dot(r * r, m16, preferred_element_type=jnp.float32) - mu1 * mu1
        h1 = (r - mu1) * lax.rsqrt(va1 + _EPS) * g1_ref[...] + b1_ref[...]
        h1 = jnp.maximum(h1, 0.0)
        h1 = jnp.dot(h1, bd1, preferred_element_type=jnp.float32) + bw1_ref[...]
        mu2 = jnp.dot(h1, m16, preferred_element_type=jnp.float32)
        va2 = jnp.dot(h1 * h1, m16, preferred_element_type=jnp.float32) - mu2 * mu2
        h2 = (h1 - mu2) * lax.rsqrt(va2 + _EPS) * g2_ref[...] + b2_ref[...]
        h2 = jnp.maximum(h2, 0.0)
        wm = jnp.dot(h2, bd2m, preferred_element_type=jnp.float32) + c2_ref[...]
        wms.append(wm)
        vals.append(vg_ref[k] + pr)

    wmax = functools.reduce(jnp.maximum, wms)
    es = [jnp.exp(w - wmax) for w in wms]
    denom = functools.reduce(jnp.add, es)
    acc = functools.reduce(jnp.add, [e * v for e, v in zip(es, vals)])
    out_ref[...] = acc / denom


def _mlp(xq, ppad, pg, kg, vg, consts):
    n = xq.shape[0]
    grid = (n // _MLP_BR,)
    small_specs = []
    for c in consts:
        small_specs.append(
            pl.BlockSpec(c.shape, lambda i, r=len(c.shape): (0,) * r))
    return pl.pallas_call(
        _mlp_body,
        grid=grid,
        in_specs=[
            pl.BlockSpec((_MLP_BR, _COUT), lambda i: (i, 0)),
            pl.BlockSpec((_MLP_BR, 16), lambda i: (i, 0)),
            pl.BlockSpec((_NS, _MLP_BR, 16), lambda i: (0, i, 0)),
            pl.BlockSpec((_NS, _MLP_BR, _COUT), lambda i: (0, i, 0)),
            pl.BlockSpec((_NS, _MLP_BR, _COUT), lambda i: (0, i, 0)),
        ] + small_specs,
        out_specs=pl.BlockSpec((_MLP_BR, _COUT), lambda i: (i, 0)),
        out_shape=jax.ShapeDtypeStruct((n, _COUT), jnp.float32),
    )(xq, ppad, pg, kg, vg, *consts)


def kernel(p, x, o, Wq, bq, Wk, bk, Wv, bv, Wp1, bp1, gp, betap, Wp2, bp2,
           gw1, betaw1, Ww1, bw1, gw2, betaw2, Ww2, bw2):
    n = x.shape[0]

    wall = jnp.concatenate([Wq, Wk, Wv], axis=1)
    ball = jnp.concatenate([bq, bk, bv])[None, :]
    ppad = jnp.pad(p, ((0, 0), (0, 13)))

    pn = jnp.sum(p * p, axis=1, keepdims=True)
    ones = jnp.ones((n, 1), jnp.float32)
    zeros = jnp.zeros((n, 3), jnp.float32)
    u = jnp.concatenate([p, pn, ones, zeros], axis=1)
    vt = jnp.concatenate([-2.0 * p, ones, pn, zeros], axis=1).T

    eye8 = jnp.eye(_HEADS, dtype=jnp.float32)
    m16 = jnp.kron(eye8, jnp.full((_MID, _MID), 1.0 / _MID, jnp.float32))
    bd1 = jnp.kron(eye8, Ww1)
    bd2 = jnp.kron(eye8, Ww2)
    bd2m = jnp.dot(bd2, m16)
    w1p = jnp.zeros((16, 16), jnp.float32).at[:3, :3].set(Wp1)
    bpp1 = jnp.zeros((1, 16), jnp.float32).at[0, :3].set(bp1)
    m3 = jnp.zeros((16, 16), jnp.float32).at[:3, :3].set(
        jnp.full((3, 3), 1.0 / 3.0, jnp.float32))
    gpp = jnp.zeros((1, 16), jnp.float32).at[0, :3].set(gp)
    bpp = jnp.zeros((1, 16), jnp.float32).at[0, :3].set(betap)
    w2p = jnp.zeros((16, _COUT), jnp.float32).at[:3, :].set(Wp2)
    bp2v = bp2[None, :]
    g1t = jnp.tile(gw1, _HEADS)[None, :]
    b1t = jnp.tile(betaw1, _HEADS)[None, :]
    bw1t = jnp.tile(bw1, _HEADS)[None, :]
    g2t = jnp.tile(gw2, _HEADS)[None, :]
    b2t = jnp.tile(betaw2, _HEADS)[None, :]
    c2 = jnp.dot(jnp.tile(bw2, _HEADS)[None, :], m16)

    xq, xk, xv = _qkv(x, wall, ball)
    idx = _knn(u, vt)
    idx_nm = idx.T.reshape(-1)
    pg, kg, vg = _sc_gather(ppad, xk, xv, idx_nm)
    pg = pg.reshape(_NS, n, 16)
    kg = kg.reshape(_NS, n, _COUT)
    vg = vg.reshape(_NS, n, _COUT)

    consts = [w1p, bpp1, m3, gpp, bpp, w2p, bp2v, m16, g1t, b1t, bd1, bw1t,
              g2t, b2t, bd2m, c2]
    out = _mlp(xq, ppad, pg, kg, vg, consts)
    return out


# speedup vs baseline: 12.7107x; 1.5874x over previous
import functools

import jax
import jax.numpy as jnp
from jax import lax
from jax.experimental import pallas as pl
from jax.experimental.pallas import tpu as pltpu
from jax.experimental.pallas import tpu_sc as plsc

_N = 8192
_NB = 4
_SEG = _N // _NB
_CIN = 128
_COUT = 128
_HEADS = 8
_MID = _COUT // _HEADS
_NS = 16
_EPS = 1e-5

_QKV_BR = 1024


def _qkv_body(x_ref, w_ref, b_ref, xq_ref, xk_ref, xv_ref):
    acc = jnp.dot(x_ref[...], w_ref[...], preferred_element_type=jnp.float32)
    acc = acc + b_ref[...]
    xq_ref[...] = acc[:, 0:_COUT]
    xk_ref[...] = acc[:, _COUT:2 * _COUT]
    xv_ref[...] = acc[:, 2 * _COUT:3 * _COUT]


def _qkv(x, wall, ball):
    n = x.shape[0]
    grid = (n // _QKV_BR,)
    return pl.pallas_call(
        _qkv_body,
        grid=grid,
        in_specs=[
            pl.BlockSpec((_QKV_BR, _CIN), lambda i: (i, 0)),
            pl.BlockSpec((_CIN, 3 * _COUT), lambda i: (0, 0)),
            pl.BlockSpec((1, 3 * _COUT), lambda i: (0, 0)),
        ],
        out_specs=[
            pl.BlockSpec((_QKV_BR, _COUT), lambda i: (i, 0)),
            pl.BlockSpec((_QKV_BR, _COUT), lambda i: (i, 0)),
            pl.BlockSpec((_QKV_BR, _COUT), lambda i: (i, 0)),
        ],
        out_shape=[jax.ShapeDtypeStruct((n, _COUT), jnp.float32)] * 3,
    )(x, wall, ball)


_KNN_BR = 256


def _knn_body(u_ref, vt_ref, idx_ref):
    s = pl.program_id(0)
    d = jnp.dot(u_ref[...], vt_ref[...], preferred_element_type=jnp.float32,
                precision=lax.Precision.HIGHEST)
    iota = lax.broadcasted_iota(jnp.int32, (_KNN_BR, _SEG), 1)
    bits = lax.bitcast_convert_type(jnp.maximum(d, 0.0), jnp.int32)
    key = (bits & jnp.int32(~2047)) | iota
    cols = []
    big = jnp.int32(0x7FFFFFFF)
    for _ in range(_NS):
        m = jnp.min(key, axis=1, keepdims=True)
        cols.append(m & jnp.int32(2047))
        key = jnp.where(key == m, big, key)
    idx_ref[...] = jnp.concatenate(cols, axis=1) + s * _SEG


def _knn(u, vt):
    grid = (_NB, _SEG // _KNN_BR)
    return pl.pallas_call(
        _knn_body,
        grid=grid,
        in_specs=[
            pl.BlockSpec((_KNN_BR, 8), lambda s, r: (s * (_SEG // _KNN_BR) + r, 0)),
            pl.BlockSpec((8, _SEG), lambda s, r: (0, s)),
        ],
        out_specs=pl.BlockSpec(
            (_KNN_BR, _NS), lambda s, r: (s * (_SEG // _KNN_BR) + r, 0)),
        out_shape=jax.ShapeDtypeStruct((_N, _NS), jnp.int32),
    )(u, vt)


_GCH = 128


def _sc_gather(ppad, xk, xv, idx_flat):
    b_total = idx_flat.shape[0]
    nw = 32
    b_per_w = b_total // nw
    nchunks = b_per_w // _GCH
    mesh = plsc.VectorSubcoreMesh(core_axis_name="c", subcore_axis_name="s")

    @functools.partial(
        pl.kernel,
        out_type=[
            jax.ShapeDtypeStruct((b_total, 16), jnp.float32),
            jax.ShapeDtypeStruct((b_total, _COUT), jnp.float32),
            jax.ShapeDtypeStruct((b_total, _COUT), jnp.float32),
        ],
        mesh=mesh,
        compiler_params=pltpu.CompilerParams(use_tc_tiling_on_sc=False),
        scratch_types=[
            pltpu.VMEM((_GCH,), jnp.int32),
            pltpu.VMEM((_GCH, 16), jnp.float32),
            pltpu.VMEM((_GCH, _COUT), jnp.float32),
            pltpu.VMEM((_GCH, _COUT), jnp.float32),
            pltpu.SemaphoreType.DMA,
            pltpu.SemaphoreType.DMA,
            pltpu.SemaphoreType.DMA,
        ],
    )
    def gather_k(ppad_hbm, xk_hbm, xv_hbm, idx_hbm, pg_out, kg_out, vg_out,
                 idx_v, pbuf, kbuf, vbuf, sem_p, sem_k, sem_v):
        wid = lax.axis_index("s") * 2 + lax.axis_index("c")
        base = pl.multiple_of(wid * b_per_w, 8)

        def chunk(ci, carry):
            off = pl.multiple_of(base + ci * _GCH, 8)
            pltpu.sync_copy(idx_hbm.at[pl.ds(off, _GCH)], idx_v)
            cp = pltpu.async_copy(ppad_hbm.at[idx_v], pbuf, sem_p)
            ck = pltpu.async_copy(xk_hbm.at[idx_v], kbuf, sem_k)
            cv = pltpu.async_copy(xv_hbm.at[idx_v], vbuf, sem_v)
            cp.wait()
            ck.wait()
            cv.wait()
            pltpu.sync_copy(pbuf, pg_out.at[pl.ds(off, _GCH)])
            pltpu.sync_copy(kbuf, kg_out.at[pl.ds(off, _GCH)])
            pltpu.sync_copy(vbuf, vg_out.at[pl.ds(off, _GCH)])
            return carry

        lax.fori_loop(0, nchunks, chunk, 0)

    return gather_k(ppad, xk, xv, idx_flat)


_MLP_BR = 256


def _mlp_body(xq_ref, pp_ref, pg_ref, kg_ref, vg_ref,
              w1p_ref, bpp1_ref, m3_ref, gpp_ref, bpp_ref, w2p_ref, bp2_ref,
              m16_ref, g1_ref, b1_ref, bd1_ref, bw1_ref,
              g2_ref, b2_ref, bd2m_ref, c2_ref, out_ref):
    f = _NS * _MLP_BR
    xq = jnp.broadcast_to(xq_ref[...][None], (_NS, _MLP_BR, _COUT)).reshape(f, _COUT)
    pp = jnp.broadcast_to(pp_ref[...][None], (_NS, _MLP_BR, 16)).reshape(f, 16)
    w1p = w1p_ref[...]
    m3 = m3_ref[...]
    w2p = w2p_ref[...]
    m16 = m16_ref[...]
    bd1 = bd1_ref[...]
    bd2m = bd2m_ref[...]

    rel = pg_ref[...].reshape(f, 16) - pp
    a = jnp.dot(rel, w1p, preferred_element_type=jnp.float32) + bpp1_ref[...]
    mu = jnp.dot(a, m3, preferred_element_type=jnp.float32)
    va = jnp.dot(a * a, m3, preferred_element_type=jnp.float32) - mu * mu
    h = (a - mu) * lax.rsqrt(va + _EPS) * gpp_ref[...] + bpp_ref[...]
    h = jnp.maximum(h, 0.0)
    pr = jnp.dot(h, w2p, preferred_element_type=jnp.float32) + bp2_ref[...]

    r = kg_ref[...].reshape(f, _COUT) + pr - xq
    mu1 = jnp.dot(r, m16, preferred_element_type=jnp.float32)
    va1 = jnp.dot(r * r, m16, preferred_element_type=jnp.float32) - mu1 * mu1
    h1 = (r - mu1) * lax.rsqrt(va1 + _EPS) * g1_ref[...] + b1_ref[...]
    h1 = jnp.maximum(h1, 0.0)
    h1 = jnp.dot(h1, bd1, preferred_element_type=jnp.float32) + bw1_ref[...]
    mu2 = jnp.dot(h1, m16, preferred_element_type=jnp.float32)
    va2 = jnp.dot(h1 * h1, m16, preferred_element_type=jnp.float32) - mu2 * mu2
    h2 = (h1 - mu2) * lax.rsqrt(va2 + _EPS) * g2_ref[...] + b2_ref[...]
    h2 = jnp.maximum(h2, 0.0)
    wm = jnp.dot(h2, bd2m, preferred_element_type=jnp.float32) + c2_ref[...]
    val = vg_ref[...].reshape(f, _COUT) + pr

    wm3 = wm.reshape(_NS, _MLP_BR, _COUT)
    val3 = val.reshape(_NS, _MLP_BR, _COUT)
    wmax = jnp.max(wm3, axis=0)
    e3 = jnp.exp(wm3 - wmax[None])
    denom = jnp.sum(e3, axis=0)
    acc = jnp.sum(e3 * val3, axis=0)
    out_ref[...] = acc / denom


def _mlp(xq, ppad, pg, kg, vg, consts):
    n = xq.shape[0]
    grid = (n // _MLP_BR,)
    small_specs = []
    for c in consts:
        small_specs.append(
            pl.BlockSpec(c.shape, lambda i, r=len(c.shape): (0,) * r))
    return pl.pallas_call(
        _mlp_body,
        grid=grid,
        in_specs=[
            pl.BlockSpec((_MLP_BR, _COUT), lambda i: (i, 0)),
            pl.BlockSpec((_MLP_BR, 16), lambda i: (i, 0)),
            pl.BlockSpec((_NS, _MLP_BR, 16), lambda i: (0, i, 0)),
            pl.BlockSpec((_NS, _MLP_BR, _COUT), lambda i: (0, i, 0)),
            pl.BlockSpec((_NS, _MLP_BR, _COUT), lambda i: (0, i, 0)),
        ] + small_specs,
        out_specs=pl.BlockSpec((_MLP_BR, _COUT), lambda i: (i, 0)),
        out_shape=jax.ShapeDtypeStruct((n, _COUT), jnp.float32),
    )(xq, ppad, pg, kg, vg, *consts)


def kernel(p, x, o, Wq, bq, Wk, bk, Wv, bv, Wp1, bp1, gp, betap, Wp2, bp2,
           gw1, betaw1, Ww1, bw1, gw2, betaw2, Ww2, bw2):
    n = x.shape[0]

    wall = jnp.concatenate([Wq, Wk, Wv], axis=1)
    ball = jnp.concatenate([bq, bk, bv])[None, :]
    ppad = jnp.pad(p, ((0, 0), (0, 13)))

    pn = jnp.sum(p * p, axis=1, keepdims=True)
    ones = jnp.ones((n, 1), jnp.float32)
    zeros = jnp.zeros((n, 3), jnp.float32)
    u = jnp.concatenate([p, pn, ones, zeros], axis=1)
    vt = jnp.concatenate([-2.0 * p, ones, pn, zeros], axis=1).T

    eye8 = jnp.eye(_HEADS, dtype=jnp.float32)
    m16 = jnp.kron(eye8, jnp.full((_MID, _MID), 1.0 / _MID, jnp.float32))
    bd1 = jnp.kron(eye8, Ww1)
    bd2 = jnp.kron(eye8, Ww2)
    bd2m = jnp.dot(bd2, m16)
    w1p = jnp.zeros((16, 16), jnp.float32).at[:3, :3].set(Wp1)
    bpp1 = jnp.zeros((1, 16), jnp.float32).at[0, :3].set(bp1)
    m3 = jnp.zeros((16, 16), jnp.float32).at[:3, :3].set(
        jnp.full((3, 3), 1.0 / 3.0, jnp.float32))
    gpp = jnp.zeros((1, 16), jnp.float32).at[0, :3].set(gp)
    bpp = jnp.zeros((1, 16), jnp.float32).at[0, :3].set(betap)
    w2p = jnp.zeros((16, _COUT), jnp.float32).at[:3, :].set(Wp2)
    bp2v = bp2[None, :]
    g1t = jnp.tile(gw1, _HEADS)[None, :]
    b1t = jnp.tile(betaw1, _HEADS)[None, :]
    bw1t = jnp.tile(bw1, _HEADS)[None, :]
    g2t = jnp.tile(gw2, _HEADS)[None, :]
    b2t = jnp.tile(betaw2, _HEADS)[None, :]
    c2 = jnp.dot(jnp.tile(bw2, _HEADS)[None, :], m16)

    xq, xk, xv = _qkv(x, wall, ball)
    idx = _knn(u, vt)
    idx_nm = idx.T.reshape(-1)
    pg, kg, vg = _sc_gather(ppad, xk, xv, idx_nm)
    pg = pg.reshape(_NS, n, 16)
    kg = kg.reshape(_NS, n, _COUT)
    vg = vg.reshape(_NS, n, _COUT)

    consts = [w1p, bpp1, m3, gpp, bpp, w2p, bp2v, m16, g1t, b1t, bd1, bw1t,
              g2t, b2t, bd2m, c2]
    out = _mlp(xq, ppad, pg, kg, vg, consts)
    return out


# speedup vs baseline: 14.3723x; 1.1307x over previous
import functools

import jax
import jax.numpy as jnp
from jax import lax
from jax.experimental import pallas as pl
from jax.experimental.pallas import tpu as pltpu
from jax.experimental.pallas import tpu_sc as plsc

_N = 8192
_NB = 4
_SEG = _N // _NB
_CIN = 128
_COUT = 128
_HEADS = 8
_MID = _COUT // _HEADS
_NS = 16
_EPS = 1e-5

_QKV_BR = 1024


def _qkv_body(x_ref, w_ref, b_ref, xq_ref, xk_ref, xv_ref):
    acc = jnp.dot(x_ref[...], w_ref[...], preferred_element_type=jnp.float32)
    acc = acc + b_ref[...]
    xq_ref[...] = acc[:, 0:_COUT]
    xk_ref[...] = acc[:, _COUT:2 * _COUT]
    xv_ref[...] = acc[:, 2 * _COUT:3 * _COUT]


def _qkv(x, wall, ball):
    n = x.shape[0]
    grid = (n // _QKV_BR,)
    return pl.pallas_call(
        _qkv_body,
        grid=grid,
        in_specs=[
            pl.BlockSpec((_QKV_BR, _CIN), lambda i: (i, 0)),
            pl.BlockSpec((_CIN, 3 * _COUT), lambda i: (0, 0)),
            pl.BlockSpec((1, 3 * _COUT), lambda i: (0, 0)),
        ],
        out_specs=[
            pl.BlockSpec((_QKV_BR, _COUT), lambda i: (i, 0)),
            pl.BlockSpec((_QKV_BR, _COUT), lambda i: (i, 0)),
            pl.BlockSpec((_QKV_BR, _COUT), lambda i: (i, 0)),
        ],
        out_shape=[jax.ShapeDtypeStruct((n, _COUT), jnp.float32)] * 3,
    )(x, wall, ball)


_KNN_BR = 256


def _knn_body(v_ref, ut_ref, idx_ref):
    s = pl.program_id(0)
    d = jnp.dot(v_ref[...], ut_ref[...], preferred_element_type=jnp.float32,
                precision=lax.Precision.HIGHEST)
    iota = lax.broadcasted_iota(jnp.int32, (_SEG, _KNN_BR), 0)
    bits = lax.bitcast_convert_type(jnp.maximum(d, 0.0), jnp.int32)
    key = (bits & jnp.int32(~2047)) | iota
    rows = []
    big = jnp.int32(0x7FFFFFFF)
    for _ in range(_NS):
        m = jnp.min(key, axis=0, keepdims=True)
        rows.append(m & jnp.int32(2047))
        key = jnp.where(key == m, big, key)
    idx_ref[...] = jnp.concatenate(rows, axis=0) + s * _SEG


def _knn(v, ut):
    grid = (_NB, _SEG // _KNN_BR)
    return pl.pallas_call(
        _knn_body,
        grid=grid,
        in_specs=[
            pl.BlockSpec((_SEG, 8), lambda s, r: (s, 0)),
            pl.BlockSpec((8, _KNN_BR), lambda s, r: (0, s * (_SEG // _KNN_BR) + r)),
        ],
        out_specs=pl.BlockSpec(
            (_NS, _KNN_BR), lambda s, r: (0, s * (_SEG // _KNN_BR) + r)),
        out_shape=jax.ShapeDtypeStruct((_NS, _N), jnp.int32),
    )(v, ut)


_GCH = 128


def _sc_gather(ppad, xk, xv, idx_flat):
    b_total = idx_flat.shape[0]
    nw = 32
    b_per_w = b_total // nw
    nchunks = b_per_w // _GCH
    mesh = plsc.VectorSubcoreMesh(core_axis_name="c", subcore_axis_name="s")

    @functools.partial(
        pl.kernel,
        out_type=[
            jax.ShapeDtypeStruct((b_total, 16), jnp.float32),
            jax.ShapeDtypeStruct((b_total, _COUT), jnp.float32),
            jax.ShapeDtypeStruct((b_total, _COUT), jnp.float32),
        ],
        mesh=mesh,
        compiler_params=pltpu.CompilerParams(use_tc_tiling_on_sc=False),
        scratch_types=[
            pltpu.VMEM((2, _GCH), jnp.int32),
            pltpu.VMEM((2, _GCH, 16), jnp.float32),
            pltpu.VMEM((2, _GCH, _COUT), jnp.float32),
            pltpu.VMEM((2, _GCH, _COUT), jnp.float32),
            pltpu.SemaphoreType.DMA,
            pltpu.SemaphoreType.DMA,
            pltpu.SemaphoreType.DMA,
            pltpu.SemaphoreType.DMA,
        ],
    )
    def gather_k(ppad_hbm, xk_hbm, xv_hbm, idx_hbm, pg_out, kg_out, vg_out,
                 idx_v, pbuf, kbuf, vbuf, sem_g0, sem_g1, sem_w0, sem_w1):
        wid = lax.axis_index("s") * 2 + lax.axis_index("c")
        base = pl.multiple_of(wid * b_per_w, 8)
        sem_g = (sem_g0, sem_g1)
        sem_w = (sem_w0, sem_w1)

        def fire(ci, b):
            off = pl.multiple_of(base + ci * _GCH, 8)
            pltpu.sync_copy(idx_hbm.at[pl.ds(off, _GCH)], idx_v.at[b])
            pltpu.async_copy(ppad_hbm.at[idx_v.at[b]], pbuf.at[b], sem_g[b])
            pltpu.async_copy(xk_hbm.at[idx_v.at[b]], kbuf.at[b], sem_g[b])
            pltpu.async_copy(xv_hbm.at[idx_v.at[b]], vbuf.at[b], sem_g[b])

        def drain_gather(b):
            pltpu.make_async_copy(ppad_hbm.at[idx_v.at[b]], pbuf.at[b], sem_g[b]).wait()
            pltpu.make_async_copy(xk_hbm.at[idx_v.at[b]], kbuf.at[b], sem_g[b]).wait()
            pltpu.make_async_copy(xv_hbm.at[idx_v.at[b]], vbuf.at[b], sem_g[b]).wait()

        def write(ci, b):
            off = pl.multiple_of(base + ci * _GCH, 8)
            pltpu.async_copy(pbuf.at[b], pg_out.at[pl.ds(off, _GCH)], sem_w[b])
            pltpu.async_copy(kbuf.at[b], kg_out.at[pl.ds(off, _GCH)], sem_w[b])
            pltpu.async_copy(vbuf.at[b], vg_out.at[pl.ds(off, _GCH)], sem_w[b])

        def drain_write(ci, b):
            off = pl.multiple_of(base + ci * _GCH, 8)
            pltpu.make_async_copy(pbuf.at[b], pg_out.at[pl.ds(off, _GCH)], sem_w[b]).wait()
            pltpu.make_async_copy(kbuf.at[b], kg_out.at[pl.ds(off, _GCH)], sem_w[b]).wait()
            pltpu.make_async_copy(vbuf.at[b], vg_out.at[pl.ds(off, _GCH)], sem_w[b]).wait()

        fire(0, 0)

        def pair(cp, carry):
            ci = 2 * cp
            fire(ci + 1, 1)
            drain_gather(0)
            write(ci, 0)
            drain_gather(1)
            write(ci + 1, 1)
            drain_write(ci, 0)

            @pl.when(cp + 1 < nchunks // 2)
            def _():
                fire(ci + 2, 0)

            drain_write(ci + 1, 1)
            return carry

        lax.fori_loop(0, nchunks // 2, pair, 0)

    return gather_k(ppad, xk, xv, idx_flat)


_MLP_BR = 256


def _mlp_body(xq_ref, pp_ref, pg_ref, kg_ref, vg_ref,
              w1p_ref, bpp1_ref, m3_ref, gpp_ref, bpp_ref, w2p_ref, bp2_ref,
              m16_ref, g1_ref, b1_ref, bd1_ref, bw1_ref,
              g2_ref, b2_ref, bd2m_ref, c2_ref, out_ref):
    f = _NS * _MLP_BR
    xq = jnp.broadcast_to(xq_ref[...][None], (_NS, _MLP_BR, _COUT)).reshape(f, _COUT)
    pp = jnp.broadcast_to(pp_ref[...][None], (_NS, _MLP_BR, 16)).reshape(f, 16)
    w1p = w1p_ref[...]
    m3 = m3_ref[...]
    w2p = w2p_ref[...]
    m16 = m16_ref[...]
    bd1 = bd1_ref[...]
    bd2m = bd2m_ref[...]

    rel = pg_ref[...].reshape(f, 16) - pp
    a = jnp.dot(rel, w1p, preferred_element_type=jnp.float32) + bpp1_ref[...]
    mu = jnp.dot(a, m3, preferred_element_type=jnp.float32)
    va = jnp.dot(a * a, m3, preferred_element_type=jnp.float32) - mu * mu
    h = (a - mu) * lax.rsqrt(va + _EPS) * gpp_ref[...] + bpp_ref[...]
    h = jnp.maximum(h, 0.0)
    pr = jnp.dot(h, w2p, preferred_element_type=jnp.float32) + bp2_ref[...]

    r = kg_ref[...].reshape(f, _COUT) + pr - xq
    mu1 = jnp.dot(r, m16, preferred_element_type=jnp.float32)
    va1 = jnp.dot(r * r, m16, preferred_element_type=jnp.float32) - mu1 * mu1
    h1 = (r - mu1) * lax.rsqrt(va1 + _EPS) * g1_ref[...] + b1_ref[...]
    h1 = jnp.maximum(h1, 0.0)
    h1 = jnp.dot(h1, bd1, preferred_element_type=jnp.float32) + bw1_ref[...]
    mu2 = jnp.dot(h1, m16, preferred_element_type=jnp.float32)
    va2 = jnp.dot(h1 * h1, m16, preferred_element_type=jnp.float32) - mu2 * mu2
    h2 = (h1 - mu2) * lax.rsqrt(va2 + _EPS) * g2_ref[...] + b2_ref[...]
    h2 = jnp.maximum(h2, 0.0)
    wm = jnp.dot(h2, bd2m, preferred_element_type=jnp.float32) + c2_ref[...]
    val = vg_ref[...].reshape(f, _COUT) + pr

    wm3 = wm.reshape(_NS, _MLP_BR, _COUT)
    val3 = val.reshape(_NS, _MLP_BR, _COUT)
    wmax = jnp.max(wm3, axis=0)
    e3 = jnp.exp(wm3 - wmax[None])
    denom = jnp.sum(e3, axis=0)
    acc = jnp.sum(e3 * val3, axis=0)
    out_ref[...] = acc / denom


def _mlp(xq, ppad, pg, kg, vg, consts):
    n = xq.shape[0]
    grid = (n // _MLP_BR,)
    small_specs = []
    for c in consts:
        small_specs.append(
            pl.BlockSpec(c.shape, lambda i, r=len(c.shape): (0,) * r))
    return pl.pallas_call(
        _mlp_body,
        grid=grid,
        in_specs=[
            pl.BlockSpec((_MLP_BR, _COUT), lambda i: (i, 0)),
            pl.BlockSpec((_MLP_BR, 16), lambda i: (i, 0)),
            pl.BlockSpec((_NS, _MLP_BR, 16), lambda i: (0, i, 0)),
            pl.BlockSpec((_NS, _MLP_BR, _COUT), lambda i: (0, i, 0)),
            pl.BlockSpec((_NS, _MLP_BR, _COUT), lambda i: (0, i, 0)),
        ] + small_specs,
        out_specs=pl.BlockSpec((_MLP_BR, _COUT), lambda i: (i, 0)),
        out_shape=jax.ShapeDtypeStruct((n, _COUT), jnp.float32),
    )(xq, ppad, pg, kg, vg, *consts)


def kernel(p, x, o, Wq, bq, Wk, bk, Wv, bv, Wp1, bp1, gp, betap, Wp2, bp2,
           gw1, betaw1, Ww1, bw1, gw2, betaw2, Ww2, bw2):
    n = x.shape[0]

    wall = jnp.concatenate([Wq, Wk, Wv], axis=1)
    ball = jnp.concatenate([bq, bk, bv])[None, :]
    ppad = jnp.pad(p, ((0, 0), (0, 13)))

    pn = jnp.sum(p * p, axis=1, keepdims=True)
    ones = jnp.ones((n, 1), jnp.float32)
    zeros = jnp.zeros((n, 3), jnp.float32)
    v = jnp.concatenate([-2.0 * p, ones, pn, zeros], axis=1)
    ut = jnp.concatenate([p, pn, ones, zeros], axis=1).T

    eye8 = jnp.eye(_HEADS, dtype=jnp.float32)
    m16 = jnp.kron(eye8, jnp.full((_MID, _MID), 1.0 / _MID, jnp.float32))
    bd1 = jnp.kron(eye8, Ww1)
    bd2 = jnp.kron(eye8, Ww2)
    bd2m = jnp.dot(bd2, m16)
    w1p = jnp.zeros((16, 16), jnp.float32).at[:3, :3].set(Wp1)
    bpp1 = jnp.zeros((1, 16), jnp.float32).at[0, :3].set(bp1)
    m3 = jnp.zeros((16, 16), jnp.float32).at[:3, :3].set(
        jnp.full((3, 3), 1.0 / 3.0, jnp.float32))
    gpp = jnp.zeros((1, 16), jnp.float32).at[0, :3].set(gp)
    bpp = jnp.zeros((1, 16), jnp.float32).at[0, :3].set(betap)
    w2p = jnp.zeros((16, _COUT), jnp.float32).at[:3, :].set(Wp2)
    bp2v = bp2[None, :]
    g1t = jnp.tile(gw1, _HEADS)[None, :]
    b1t = jnp.tile(betaw1, _HEADS)[None, :]
    bw1t = jnp.tile(bw1, _HEADS)[None, :]
    g2t = jnp.tile(gw2, _HEADS)[None, :]
    b2t = jnp.tile(betaw2, _HEADS)[None, :]
    c2 = jnp.dot(jnp.tile(bw2, _HEADS)[None, :], m16)

    xq, xk, xv = _qkv(x, wall, ball)
    idx = _knn(v, ut)
    idx_nm = idx.reshape(-1)
    pg, kg, vg = _sc_gather(ppad, xk, xv, idx_nm)
    pg = pg.reshape(_NS, n, 16)
    kg = kg.reshape(_NS, n, _COUT)
    vg = vg.reshape(_NS, n, _COUT)

    consts = [w1p, bpp1, m3, gpp, bpp, w2p, bp2v, m16, g1t, b1t, bd1, bw1t,
              g2t, b2t, bd2m, c2]
    out = _mlp(xq, ppad, pg, kg, vg, consts)
    return out


# speedup vs baseline: 16.0034x; 1.1135x over previous
import functools

import jax
import jax.numpy as jnp
from jax import lax
from jax.experimental import pallas as pl
from jax.experimental.pallas import tpu as pltpu
from jax.experimental.pallas import tpu_sc as plsc

_N = 8192
_NB = 4
_SEG = _N // _NB
_CIN = 128
_COUT = 128
_HEADS = 8
_MID = _COUT // _HEADS
_NS = 16
_EPS = 1e-5

_QKV_BR = 1024


def _qkv_body(x_ref, w_ref, b_ref, xq_ref, xk_ref, xv_ref):
    acc = jnp.dot(x_ref[...], w_ref[...], preferred_element_type=jnp.float32)
    acc = acc + b_ref[...]
    xq_ref[...] = acc[:, 0:_COUT]
    xk_ref[...] = acc[:, _COUT:2 * _COUT]
    xv_ref[...] = acc[:, 2 * _COUT:3 * _COUT]


def _qkv(x, wall, ball):
    n = x.shape[0]
    grid = (n // _QKV_BR,)
    return pl.pallas_call(
        _qkv_body,
        grid=grid,
        in_specs=[
            pl.BlockSpec((_QKV_BR, _CIN), lambda i: (i, 0)),
            pl.BlockSpec((_CIN, 3 * _COUT), lambda i: (0, 0)),
            pl.BlockSpec((1, 3 * _COUT), lambda i: (0, 0)),
        ],
        out_specs=[
            pl.BlockSpec((_QKV_BR, _COUT), lambda i: (i, 0)),
            pl.BlockSpec((_QKV_BR, _COUT), lambda i: (i, 0)),
            pl.BlockSpec((_QKV_BR, _COUT), lambda i: (i, 0)),
        ],
        out_shape=[jax.ShapeDtypeStruct((n, _COUT), jnp.float32)] * 3,
    )(x, wall, ball)


_KNN_BR = 256


def _knn(v, ut, s):
    seg_start = s * _SEG
    nblk = _SEG // _KNN_BR

    def body(v_ref, ut_ref, idx_ref):
        d = jnp.dot(v_ref[...], ut_ref[...], preferred_element_type=jnp.float32,
                    precision=lax.Precision.HIGHEST)
        iota = lax.broadcasted_iota(jnp.int32, (_SEG, _KNN_BR), 0)
        bits = lax.bitcast_convert_type(jnp.maximum(d, 0.0), jnp.int32)
        key = (bits & jnp.int32(~2047)) | iota
        rows = []
        big = jnp.int32(0x7FFFFFFF)
        for _ in range(_NS):
            m = jnp.min(key, axis=0, keepdims=True)
            rows.append(m & jnp.int32(2047))
            key = jnp.where(key == m, big, key)
        idx_ref[...] = jnp.concatenate(rows, axis=0) + seg_start

    return pl.pallas_call(
        body,
        grid=(nblk,),
        in_specs=[
            pl.BlockSpec((_SEG, 8), lambda r: (s, 0)),
            pl.BlockSpec((8, _KNN_BR), lambda r: (0, s * nblk + r)),
        ],
        out_specs=pl.BlockSpec((_NS, _KNN_BR), lambda r: (0, r)),
        out_shape=jax.ShapeDtypeStruct((_NS, _SEG), jnp.int32),
    )(v, ut)


_GCH = 128


def _sc_gather(ppad, xk, xv, idx_flat):
    b_total = idx_flat.shape[0]
    nw = 32
    b_per_w = b_total // nw
    nchunks = b_per_w // _GCH
    mesh = plsc.VectorSubcoreMesh(core_axis_name="c", subcore_axis_name="s")

    @functools.partial(
        pl.kernel,
        out_type=[
            jax.ShapeDtypeStruct((b_total, 16), jnp.float32),
            jax.ShapeDtypeStruct((b_total, _COUT), jnp.float32),
            jax.ShapeDtypeStruct((b_total, _COUT), jnp.float32),
        ],
        mesh=mesh,
        compiler_params=pltpu.CompilerParams(use_tc_tiling_on_sc=False),
        scratch_types=[
            pltpu.VMEM((2, _GCH), jnp.int32),
            pltpu.VMEM((2, _GCH, 16), jnp.float32),
            pltpu.VMEM((2, _GCH, _COUT), jnp.float32),
            pltpu.VMEM((2, _GCH, _COUT), jnp.float32),
            pltpu.SemaphoreType.DMA,
            pltpu.SemaphoreType.DMA,
            pltpu.SemaphoreType.DMA,
            pltpu.SemaphoreType.DMA,
        ],
    )
    def gather_k(ppad_hbm, xk_hbm, xv_hbm, idx_hbm, pg_out, kg_out, vg_out,
                 idx_v, pbuf, kbuf, vbuf, sem_g0, sem_g1, sem_w0, sem_w1):
        wid = lax.axis_index("s") * 2 + lax.axis_index("c")
        base = pl.multiple_of(wid * b_per_w, 8)
        sem_g = (sem_g0, sem_g1)
        sem_w = (sem_w0, sem_w1)

        def fire(ci, b):
            off = pl.multiple_of(base + ci * _GCH, 8)
            pltpu.sync_copy(idx_hbm.at[pl.ds(off, _GCH)], idx_v.at[b])
            pltpu.async_copy(ppad_hbm.at[idx_v.at[b]], pbuf.at[b], sem_g[b])
            pltpu.async_copy(xk_hbm.at[idx_v.at[b]], kbuf.at[b], sem_g[b])
            pltpu.async_copy(xv_hbm.at[idx_v.at[b]], vbuf.at[b], sem_g[b])

        def drain_gather(b):
            pltpu.make_async_copy(ppad_hbm.at[idx_v.at[b]], pbuf.at[b], sem_g[b]).wait()
            pltpu.make_async_copy(xk_hbm.at[idx_v.at[b]], kbuf.at[b], sem_g[b]).wait()
            pltpu.make_async_copy(xv_hbm.at[idx_v.at[b]], vbuf.at[b], sem_g[b]).wait()

        def write(ci, b):
            off = pl.multiple_of(base + ci * _GCH, 8)
            pltpu.async_copy(pbuf.at[b], pg_out.at[pl.ds(off, _GCH)], sem_w[b])
            pltpu.async_copy(kbuf.at[b], kg_out.at[pl.ds(off, _GCH)], sem_w[b])
            pltpu.async_copy(vbuf.at[b], vg_out.at[pl.ds(off, _GCH)], sem_w[b])

        def drain_write(ci, b):
            off = pl.multiple_of(base + ci * _GCH, 8)
            pltpu.make_async_copy(pbuf.at[b], pg_out.at[pl.ds(off, _GCH)], sem_w[b]).wait()
            pltpu.make_async_copy(kbuf.at[b], kg_out.at[pl.ds(off, _GCH)], sem_w[b]).wait()
            pltpu.make_async_copy(vbuf.at[b], vg_out.at[pl.ds(off, _GCH)], sem_w[b]).wait()

        fire(0, 0)

        def pair(cp, carry):
            ci = 2 * cp
            fire(ci + 1, 1)
            drain_gather(0)
            write(ci, 0)
            drain_gather(1)
            write(ci + 1, 1)
            drain_write(ci, 0)

            @pl.when(cp + 1 < nchunks // 2)
            def _():
                fire(ci + 2, 0)

            drain_write(ci + 1, 1)
            return carry

        lax.fori_loop(0, nchunks // 2, pair, 0)

    return gather_k(ppad, xk, xv, idx_flat)


_MLP_BR = 256


def _mlp_body(xq_ref, pp_ref, pg_ref, kg_ref, vg_ref,
              w1p_ref, bpp1_ref, m3_ref, gpp_ref, bpp_ref, w2p_ref, bp2_ref,
              m16_ref, g1_ref, b1_ref, bd1_ref, bw1_ref,
              g2_ref, b2_ref, bd2m_ref, c2_ref, out_ref):
    f = _NS * _MLP_BR
    xq = jnp.broadcast_to(xq_ref[...][None], (_NS, _MLP_BR, _COUT)).reshape(f, _COUT)
    pp = jnp.broadcast_to(pp_ref[...][None], (_NS, _MLP_BR, 16)).reshape(f, 16)
    w1p = w1p_ref[...]
    m3 = m3_ref[...]
    w2p = w2p_ref[...]
    m16 = m16_ref[...]
    bd1 = bd1_ref[...]
    bd2m = bd2m_ref[...]

    rel = pg_ref[...].reshape(f, 16) - pp
    a = jnp.dot(rel, w1p, preferred_element_type=jnp.float32) + bpp1_ref[...]
    mu = jnp.dot(a, m3, preferred_element_type=jnp.float32)
    va = jnp.dot(a * a, m3, preferred_element_type=jnp.float32) - mu * mu
    h = (a - mu) * lax.rsqrt(va + _EPS) * gpp_ref[...] + bpp_ref[...]
    h = jnp.maximum(h, 0.0)
    pr = jnp.dot(h, w2p, preferred_element_type=jnp.float32) + bp2_ref[...]

    r = kg_ref[...].reshape(f, _COUT) + pr - xq
    mu1 = jnp.dot(r, m16, preferred_element_type=jnp.float32)
    va1 = jnp.dot(r * r, m16, preferred_element_type=jnp.float32) - mu1 * mu1
    h1 = (r - mu1) * lax.rsqrt(va1 + _EPS) * g1_ref[...] + b1_ref[...]
    h1 = jnp.maximum(h1, 0.0)
    h1 = jnp.dot(h1, bd1, preferred_element_type=jnp.float32) + bw1_ref[...]
    mu2 = jnp.dot(h1, m16, preferred_element_type=jnp.float32)
    va2 = jnp.dot(h1 * h1, m16, preferred_element_type=jnp.float32) - mu2 * mu2
    h2 = (h1 - mu2) * lax.rsqrt(va2 + _EPS) * g2_ref[...] + b2_ref[...]
    h2 = jnp.maximum(h2, 0.0)
    wm = jnp.dot(h2, bd2m, preferred_element_type=jnp.float32) + c2_ref[...]
    val = vg_ref[...].reshape(f, _COUT) + pr

    wm3 = wm.reshape(_NS, _MLP_BR, _COUT)
    val3 = val.reshape(_NS, _MLP_BR, _COUT)
    wmax = jnp.max(wm3, axis=0)
    e3 = jnp.exp(wm3 - wmax[None])
    denom = jnp.sum(e3, axis=0)
    acc = jnp.sum(e3 * val3, axis=0)
    out_ref[...] = acc / denom


def _mlp(xq, ppad, pg, kg, vg, consts, s):
    soff = s * (_SEG // _MLP_BR)
    grid = (_SEG // _MLP_BR,)
    small_specs = []
    for c in consts:
        small_specs.append(
            pl.BlockSpec(c.shape, lambda i, r=len(c.shape): (0,) * r))
    return pl.pallas_call(
        _mlp_body,
        grid=grid,
        in_specs=[
            pl.BlockSpec((_MLP_BR, _COUT), lambda i: (soff + i, 0)),
            pl.BlockSpec((_MLP_BR, 16), lambda i: (soff + i, 0)),
            pl.BlockSpec((_NS, _MLP_BR, 16), lambda i: (0, i, 0)),
            pl.BlockSpec((_NS, _MLP_BR, _COUT), lambda i: (0, i, 0)),
            pl.BlockSpec((_NS, _MLP_BR, _COUT), lambda i: (0, i, 0)),
        ] + small_specs,
        out_specs=pl.BlockSpec((_MLP_BR, _COUT), lambda i: (i, 0)),
        out_shape=jax.ShapeDtypeStruct((_SEG, _COUT), jnp.float32),
    )(xq, ppad, pg, kg, vg, *consts)


def kernel(p, x, o, Wq, bq, Wk, bk, Wv, bv, Wp1, bp1, gp, betap, Wp2, bp2,
           gw1, betaw1, Ww1, bw1, gw2, betaw2, Ww2, bw2):
    n = x.shape[0]

    wall = jnp.concatenate([Wq, Wk, Wv], axis=1)
    ball = jnp.concatenate([bq, bk, bv])[None, :]
    ppad = jnp.pad(p, ((0, 0), (0, 13)))

    pn = jnp.sum(p * p, axis=1, keepdims=True)
    ones = jnp.ones((n, 1), jnp.float32)
    zeros = jnp.zeros((n, 3), jnp.float32)
    v = jnp.concatenate([-2.0 * p, ones, pn, zeros], axis=1)
    ut = jnp.concatenate([p, pn, ones, zeros], axis=1).T

    eye8 = jnp.eye(_HEADS, dtype=jnp.float32)
    m16 = jnp.kron(eye8, jnp.full((_MID, _MID), 1.0 / _MID, jnp.float32))
    bd1 = jnp.kron(eye8, Ww1)
    bd2 = jnp.kron(eye8, Ww2)
    bd2m = jnp.dot(bd2, m16)
    w1p = jnp.zeros((16, 16), jnp.float32).at[:3, :3].set(Wp1)
    bpp1 = jnp.zeros((1, 16), jnp.float32).at[0, :3].set(bp1)
    m3 = jnp.zeros((16, 16), jnp.float32).at[:3, :3].set(
        jnp.full((3, 3), 1.0 / 3.0, jnp.float32))
    gpp = jnp.zeros((1, 16), jnp.float32).at[0, :3].set(gp)
    bpp = jnp.zeros((1, 16), jnp.float32).at[0, :3].set(betap)
    w2p = jnp.zeros((16, _COUT), jnp.float32).at[:3, :].set(Wp2)
    bp2v = bp2[None, :]
    g1t = jnp.tile(gw1, _HEADS)[None, :]
    b1t = jnp.tile(betaw1, _HEADS)[None, :]
    bw1t = jnp.tile(bw1, _HEADS)[None, :]
    g2t = jnp.tile(gw2, _HEADS)[None, :]
    b2t = jnp.tile(betaw2, _HEADS)[None, :]
    c2 = jnp.dot(jnp.tile(bw2, _HEADS)[None, :], m16)

    consts = [w1p, bpp1, m3, gpp, bpp, w2p, bp2v, m16, g1t, b1t, bd1, bw1t,
              g2t, b2t, bd2m, c2]

    xq, xk, xv = _qkv(x, wall, ball)
    outs = []
    for s in range(_NB):
        idx_s = _knn(v, ut, s)
        pg, kg, vg = _sc_gather(ppad, xk, xv, idx_s.reshape(-1))
        pg = pg.reshape(_NS, _SEG, 16)
        kg = kg.reshape(_NS, _SEG, _COUT)
        vg = vg.reshape(_NS, _SEG, _COUT)
        outs.append(_mlp(xq, ppad, pg, kg, vg, consts, s))
    return jnp.concatenate(outs, axis=0)
